# baseline (speedup 1.0000x reference)
import jax
import jax.numpy as jnp
from jax import lax
from jax.experimental import pallas as pl
from jax.experimental.pallas import tpu as pltpu

D_MODEL = 768
PE_PAD = 64
BLK = 4096


def _tc_body(idx_ref, x_ref, pe_ref, o_ref):
    idx = idx_ref[0, 0, :]
    oh = (idx[:, None] == lax.broadcasted_iota(jnp.int32, (BLK, PE_PAD), 1))
    gathered = jnp.dot(
        oh.astype(jnp.float32), pe_ref[...], preferred_element_type=jnp.float32
    )
    o_ref[...] = x_ref[...] + gathered


@jax.jit
def _tc_add_pe(x2d, idx, pe_pad):
    rows = x2d.shape[0]
    n = rows // BLK
    idx3 = idx.reshape(n, 1, BLK)
    return pl.pallas_call(
        _tc_body,
        grid=(n,),
        in_specs=[
            pl.BlockSpec((1, 1, BLK), lambda i: (i, 0, 0)),
            pl.BlockSpec((BLK, D_MODEL), lambda i: (i, 0)),
            pl.BlockSpec((PE_PAD, D_MODEL), lambda i: (0, 0)),
        ],
        out_specs=pl.BlockSpec((BLK, D_MODEL), lambda i: (i, 0)),
        out_shape=jax.ShapeDtypeStruct((rows, D_MODEL), jnp.float32),
    )(idx3, x2d, pe_pad)


def kernel(x, created_list, pe):
    b, s, d = x.shape
    rows = b * s
    x2d = x.reshape(rows, d)
    idx = created_list.reshape(rows).astype(jnp.int32)
    pe2d = pe.reshape(pe.shape[0], d)
    pe_pad = jnp.pad(pe2d, ((0, PE_PAD - pe2d.shape[0]), (0, 0)))
    out = _tc_add_pe(x2d, idx, pe_pad)
    return out.reshape(b, s, d)
